# Initial kernel scaffold; baseline (speedup 1.0000x reference)
import jax
import jax.numpy as jnp
from jax.experimental import pallas as pl


def kernel(node_id, edge_index, edge_type, emb, W0, Ws0, b0, W1, Ws1, b1):
    raise NotImplementedError("write your pallas kernel here")


# speedup vs baseline: 1.5286x; 1.5286x over previous
import functools

import jax
import jax.numpy as jnp
from jax import lax
from jax.experimental import pallas as pl
from jax.experimental.pallas import tpu as pltpu
from jax.experimental.pallas import tpu_sc as plsc

N = 10000
E = 320000
R = 16
H = 128

NC = 2
NS = 16
NW = NC * NS
CHUNK = 128
CPS = 2
STEPS = 80 // CPS
CHUNKS_PER_TILE = 80
E_PAD = NW * CHUNKS_PER_TILE * CHUNK
N_PAD = 10240
STRIPE = N_PAD // NS

BN = 1000


def _transform_body(h_ref, w_ref, ws_ref, b_ref, tr_ref, sh_ref):
    hb = h_ref[...]
    for r in range(R):
        tr_ref[r] = jnp.dot(hb, w_ref[r], preferred_element_type=jnp.float32)
    sh_ref[...] = (
        jnp.dot(hb, ws_ref[...], preferred_element_type=jnp.float32) + b_ref[0]
    )


def _combine_transform_body(p_ref, sh_in_ref, w_ref, ws_ref, b_ref,
                            tr_ref, sh_ref):
    hb = p_ref[0] + p_ref[1] + sh_in_ref[...]
    for r in range(R):
        tr_ref[r] = jnp.dot(hb, w_ref[r], preferred_element_type=jnp.float32)
    sh_ref[...] = (
        jnp.dot(hb, ws_ref[...], preferred_element_type=jnp.float32) + b_ref[0]
    )


def _final_body(p_ref, sh_ref, out_ref):
    out_ref[...] = p_ref[0] + p_ref[1] + sh_ref[...]


def _tc_transform(h, w, ws, b2):
    return pl.pallas_call(
        _transform_body,
        grid=(N // BN,),
        in_specs=[
            pl.BlockSpec((BN, H), lambda n: (n, 0)),
            pl.BlockSpec((R, H, H), lambda n: (0, 0, 0)),
            pl.BlockSpec((H, H), lambda n: (0, 0)),
            pl.BlockSpec((1, H), lambda n: (0, 0)),
        ],
        out_specs=[
            pl.BlockSpec((R, BN, H), lambda n: (0, n, 0)),
            pl.BlockSpec((BN, H), lambda n: (n, 0)),
        ],
        out_shape=[
            jax.ShapeDtypeStruct((R, N, H), jnp.float32),
            jax.ShapeDtypeStruct((N, H), jnp.float32),
        ],
    )(h, w, ws, b2)


def _tc_combine_transform(p, sh_in, w, ws, b2):
    return pl.pallas_call(
        _combine_transform_body,
        grid=(N // BN,),
        in_specs=[
            pl.BlockSpec((2, BN, H), lambda n: (0, n, 0)),
            pl.BlockSpec((BN, H), lambda n: (n, 0)),
            pl.BlockSpec((R, H, H), lambda n: (0, 0, 0)),
            pl.BlockSpec((H, H), lambda n: (0, 0)),
            pl.BlockSpec((1, H), lambda n: (0, 0)),
        ],
        out_specs=[
            pl.BlockSpec((R, BN, H), lambda n: (0, n, 0)),
            pl.BlockSpec((BN, H), lambda n: (n, 0)),
        ],
        out_shape=[
            jax.ShapeDtypeStruct((R, N, H), jnp.float32),
            jax.ShapeDtypeStruct((N, H), jnp.float32),
        ],
    )(p, sh_in, w, ws, b2)


def _tc_final(p, sh):
    return pl.pallas_call(
        _final_body,
        grid=(N // BN,),
        in_specs=[
            pl.BlockSpec((2, BN, H), lambda n: (0, n, 0)),
            pl.BlockSpec((BN, H), lambda n: (n, 0)),
        ],
        out_specs=pl.BlockSpec((BN, H), lambda n: (n, 0)),
        out_shape=jax.ShapeDtypeStruct((N, H), jnp.float32),
    )(p, sh)


def _sc_aggregate_body(tr_hbm, idx_hbm, dst_hbm, zz_hbm, out_hbm,
                       acc, idxv, dstv, rows, sem):
    c = lax.axis_index("c")
    s = lax.axis_index("s")
    wid = s * NC + c

    pltpu.sync_copy(zz_hbm, acc.at[pl.ds(s * STRIPE, STRIPE)])
    plsc.subcore_barrier()

    base_row = wid * CHUNKS_PER_TILE

    def step(g, carry):
        row0 = base_row + g * CPS
        pltpu.sync_copy(idx_hbm.at[pl.ds(row0, CPS)], idxv)
        pltpu.sync_copy(dst_hbm.at[pl.ds(row0, CPS)], dstv)
        cps = [
            pltpu.async_copy(tr_hbm.at[idxv.at[j]], rows.at[j], sem)
            for j in range(CPS)
        ]
        for j in range(CPS):
            cps[j].wait()
            pltpu.sync_copy(rows.at[j], acc.at[dstv.at[j]], add=True)
        return carry

    lax.fori_loop(0, STEPS, step, 0)

    plsc.subcore_barrier()
    pltpu.sync_copy(acc.at[pl.ds(s * STRIPE, STRIPE)],
                    out_hbm.at[c, pl.ds(s * STRIPE, STRIPE)])


@functools.lru_cache(maxsize=1)
def _sc_aggregate_kernel():
    return pl.kernel(
        _sc_aggregate_body,
        out_type=jax.ShapeDtypeStruct((NC, N_PAD, H), jnp.float32),
        scratch_types=[
            pltpu.VMEM_SHARED((N_PAD, H), jnp.float32),
            pltpu.VMEM((CPS, CHUNK), jnp.int32),
            pltpu.VMEM((CPS, CHUNK), jnp.int32),
            pltpu.VMEM((CPS, CHUNK, H), jnp.float32),
            pltpu.SemaphoreType.DMA,
        ],
        mesh=plsc.VectorSubcoreMesh(core_axis_name="c", subcore_axis_name="s"),
    )


def _sc_aggregate(tr_flat, idx2d, dst2d, zz):
    return _sc_aggregate_kernel()(tr_flat, idx2d, dst2d, zz)


def kernel(node_id, edge_index, edge_type, emb, W0, Ws0, b0, W1, Ws1, b1):
    h0 = jnp.take(emb, node_id, axis=0)
    src = edge_index[0]
    dst = edge_index[1]

    pad = E_PAD - E
    idx = edge_type.astype(jnp.int32) * N + src.astype(jnp.int32)
    idx2d = jnp.concatenate(
        [idx, jnp.zeros((pad,), jnp.int32)]).reshape(E_PAD // CHUNK, CHUNK)
    dst2d = jnp.concatenate(
        [dst.astype(jnp.int32), jnp.full((pad,), N, jnp.int32)]
    ).reshape(E_PAD // CHUNK, CHUNK)
    zz = jnp.zeros((STRIPE, H), jnp.float32)

    b0r = b0.reshape(1, H)
    b1r = b1.reshape(1, H)

    tr0, sh0 = _tc_transform(h0, W0, Ws0, b0r)
    p0 = _sc_aggregate(tr0.reshape(R * N, H), idx2d, dst2d, zz)
    tr1, sh1 = _tc_combine_transform(p0, sh0, W1, Ws1, b1r)
    p1 = _sc_aggregate(tr1.reshape(R * N, H), idx2d, dst2d, zz)
    return _tc_final(p1, sh1)
